# baseline (speedup 1.0000x reference)
import functools

import jax
import jax.numpy as jnp
from jax import lax
from jax.experimental import pallas as pl
from jax.experimental.pallas import tpu as pltpu
from jax.experimental.pallas import tpu_sc as plsc

N = 10000
NP = 10240
E = 320000
H = 128
NT = 4
G = 64

NSC = 2
NSUB = 16
NW = NSC * NSUB
EPT = E // NW
EPTP = 10240
CH = 128
NCHUNK = EPTP // CH
NPS = NP // NSUB
NGRP = EPT // 16
NGRPP = EPTP // 16
HH = H // 2
BLK = 1024
NBLK = NP // BLK

_mesh = plsc.VectorSubcoreMesh(core_axis_name="c", subcore_axis_name="s",
                               num_cores=NSC, num_subcores=NSUB)

_IOTA = lambda: lax.broadcasted_iota(jnp.int32, (16,), 0)


def _vgather(v, i):
    return v.at[i].get(mode="promise_in_bounds")


def _seg_update(dv, val, outref, op):
    iota = _IOTA()
    ks, vs = plsc.sort_key_val(dv, val)
    for sh in (1, 2, 4, 8):
        idxm = jnp.maximum(iota - sh, 0)
        kprev = _vgather(ks, idxm)
        vprev = _vgather(vs, idxm)
        valid = (iota >= sh) & (kprev == ks)
        if op == "max":
            vs = jnp.where(valid, jnp.maximum(vs, vprev), vs)
        else:
            vs = vs + jnp.where(valid, vprev, jnp.float32(0.0))
    knext = _vgather(ks, jnp.minimum(iota + 1, 15))
    runend = (knext != ks) | (iota == 15)
    old = plsc.load_gather(outref, [ks])
    newv = jnp.maximum(old, vs) if op == "max" else old + vs
    plsc.store_scatter(outref, [ks], newv, mask=runend)


def _wid(c, s):
    return c * NSUB + s


def _tc_layer_body(first, *refs):
    if first:
        x0_ref, w_ref, asad_ref, we_ref, hwe0_ref, hwe1_ref, sd_ref, es_ref = refs
        h = x0_ref[...]
    else:
        (pa0_ref, pa1_ref, pb0_ref, pb1_ref, w_ref, asad_ref, we_ref,
         hwe0_ref, hwe1_ref, sd_ref, es_ref) = refs
        h = jnp.concatenate(
            [jnp.maximum(pa0_ref[...] + pa1_ref[...], 0.0),
             jnp.maximum(pb0_ref[...] + pb1_ref[...], 0.0)], axis=1)
    i = pl.program_id(0)
    hw = jnp.dot(h, w_ref[...], preferred_element_type=jnp.float32)
    we = we_ref[...]
    hwe0_ref[...] = hw[None, :, :HH] + we[:, None, :HH]
    hwe1_ref[...] = hw[None, :, HH:] + we[:, None, HH:]
    sd_ref[...] = jnp.dot(hw, asad_ref[...], preferred_element_type=jnp.float32)

    @pl.when(i == 0)
    def _():
        esd = jnp.dot(we_ref[...], asad_ref[...],
                      preferred_element_type=jnp.float32)
        es_ref[...] = jnp.concatenate(
            [esd, jnp.zeros((NT, 2), jnp.float32)], axis=0)


def _make_tc_layer(first):
    if first:
        in_specs = [pl.BlockSpec((BLK, H), lambda i: (i, 0))]
    else:
        in_specs = [pl.BlockSpec((BLK, HH), lambda i: (i, 0))] * 4

    def run(*args):
        return pl.pallas_call(
            functools.partial(_tc_layer_body, first),
            grid=(NBLK,),
            in_specs=in_specs + [
                pl.BlockSpec((H, H), lambda i: (0, 0)),
                pl.BlockSpec((H, 2), lambda i: (0, 0)),
                pl.BlockSpec((NT, H), lambda i: (0, 0)),
            ],
            out_specs=[
                pl.BlockSpec((NT, BLK, HH), lambda i: (0, i, 0)),
                pl.BlockSpec((NT, BLK, HH), lambda i: (0, i, 0)),
                pl.BlockSpec((BLK, 2), lambda i: (i, 0)),
                pl.BlockSpec((2 * NT, 2), lambda i: (0, 0)),
            ],
            out_shape=[
                jax.ShapeDtypeStruct((NT, NP, HH), jnp.float32),
                jax.ShapeDtypeStruct((NT, NP, HH), jnp.float32),
                jax.ShapeDtypeStruct((NP, 2), jnp.float32),
                jax.ShapeDtypeStruct((2 * NT, 2), jnp.float32),
            ],
        )(*args)

    return run


_tc_layer_first = _make_tc_layer(True)
_tc_layer_next = _make_tc_layer(False)


def _tc_head_body(pa0_ref, pa1_ref, pb0_ref, pb1_ref, b3_ref, wout_ref,
                  bout_ref, wv1_ref, bv1_ref, wf_ref, bf_ref, o_ref, g_acc):
    i = pl.program_id(0)

    @pl.when(i == 0)
    def _():
        g_acc[...] = jnp.zeros_like(g_acc)

    h = jnp.concatenate(
        [jnp.maximum(pa0_ref[...] + pa1_ref[...], 0.0),
         jnp.maximum(pb0_ref[...] + pb1_ref[...], 0.0)], axis=1)
    ho = jnp.dot(h, wout_ref[...], preferred_element_type=jnp.float32)
    ho = ho + bout_ref[0, :]
    b = b3_ref[0, 0, :]
    ids = lax.broadcasted_iota(jnp.int32, (G, BLK), 0)
    m = jnp.where(b[None, :] == ids, 1.0, 0.0)
    g_acc[...] += jnp.dot(m, ho, preferred_element_type=jnp.float32)

    @pl.when(i == NBLK - 1)
    def _():
        xx = jnp.maximum(g_acc[...], 0.0)
        xx = jnp.dot(xx, wv1_ref[...], preferred_element_type=jnp.float32)
        xx = jnp.maximum(xx + bv1_ref[0, :], 0.0)
        o_ref[...] = jnp.dot(xx, wf_ref[...],
                             preferred_element_type=jnp.float32) + bf_ref[0, :]


def _tc_head(pa0, pa1, pb0, pb1, b3, wout, bout, wv1, bv1, wfp, bfp):
    return pl.pallas_call(
        _tc_head_body,
        grid=(NBLK,),
        in_specs=[
            pl.BlockSpec((BLK, HH), lambda i: (i, 0)),
            pl.BlockSpec((BLK, HH), lambda i: (i, 0)),
            pl.BlockSpec((BLK, HH), lambda i: (i, 0)),
            pl.BlockSpec((BLK, HH), lambda i: (i, 0)),
            pl.BlockSpec((1, 1, BLK), lambda i: (i, 0, 0)),
            pl.BlockSpec((H, 256), lambda i: (0, 0)),
            pl.BlockSpec((1, 256), lambda i: (0, 0)),
            pl.BlockSpec((256, 256), lambda i: (0, 0)),
            pl.BlockSpec((1, 256), lambda i: (0, 0)),
            pl.BlockSpec((256, 8), lambda i: (0, 0)),
            pl.BlockSpec((1, 8), lambda i: (0, 0)),
        ],
        out_specs=pl.BlockSpec((G, 8), lambda i: (0, 0)),
        out_shape=jax.ShapeDtypeStruct((G, 8), jnp.float32),
        scratch_shapes=[pltpu.VMEM((G, 256), jnp.float32)],
    )(pa0, pa1, pb0, pb1, b3, wout, bout, wv1, bv1, wfp, bfp)


@functools.partial(
    pl.kernel,
    out_type=(jax.ShapeDtypeStruct((E,), jnp.float32),
              jax.ShapeDtypeStruct((NW, NP), jnp.float32)),
    mesh=_mesh,
    compiler_params=pltpu.CompilerParams(needs_layout_passes=False),
    scratch_types=[
        pltpu.VMEM((EPT,), jnp.int32),
        pltpu.VMEM((EPT,), jnp.int32),
        pltpu.VMEM((EPT,), jnp.int32),
        pltpu.VMEM((2 * NP,), jnp.float32),
        pltpu.VMEM((4 * NT,), jnp.float32),
        pltpu.VMEM((EPT,), jnp.float32),
        pltpu.VMEM((NP,), jnp.float32),
    ],
)
def _sc_a1(src_h, dst_h, et_h, sd_h, es_h, logits_h, mxp_h,
           srcb, dstb, etb, sdb, esb, logb, mxb):
    c = lax.axis_index("c")
    s = lax.axis_index("s")
    w = _wid(c, s)
    e0 = w * EPT
    pltpu.sync_copy(src_h.at[pl.ds(e0, EPT)], srcb)
    pltpu.sync_copy(dst_h.at[pl.ds(e0, EPT)], dstb)
    pltpu.sync_copy(et_h.at[pl.ds(e0, EPT)], etb)
    pltpu.sync_copy(sd_h, sdb)
    pltpu.sync_copy(es_h, esb)

    neginf = jnp.full((16,), -jnp.inf, jnp.float32)

    def init_body(i, _):
        mxb[pl.ds(i * 16, 16)] = neginf
        return 0

    lax.fori_loop(0, NP // 16, init_body, 0)

    def grp_body(g, _):
        sl = pl.ds(g * 16, 16)
        sv = srcb[sl]
        dv = dstb[sl]
        ev = etb[sl]
        s16 = plsc.load_gather(sdb, [sv * 2])
        d16 = plsc.load_gather(sdb, [dv * 2 + 1])
        e16 = plsc.load_gather(esb, [ev * 2])
        z = s16 + d16 + e16
        lg = jnp.maximum(z, 0.2 * z)
        logb[sl] = lg
        _seg_update(dv, lg, mxb, "max")
        return 0

    lax.fori_loop(0, NGRP, grp_body, 0)

    pltpu.sync_copy(logb, logits_h.at[pl.ds(e0, EPT)])
    pltpu.sync_copy(mxb, mxp_h.at[w])


@functools.partial(
    pl.kernel,
    out_type=(jax.ShapeDtypeStruct((E,), jnp.float32),
              jax.ShapeDtypeStruct((NW, NP), jnp.float32)),
    mesh=_mesh,
    compiler_params=pltpu.CompilerParams(needs_layout_passes=False),
    scratch_types=[
        pltpu.VMEM((EPT,), jnp.int32),
        pltpu.VMEM((EPT,), jnp.float32),
        pltpu.VMEM((NP,), jnp.float32),
        pltpu.VMEM((NP,), jnp.float32),
        pltpu.VMEM((NW, NPS), jnp.float32),
        pltpu.VMEM_SHARED((NP,), jnp.float32),
    ],
)
def _sc_a3(dst_h, logits_h, mxp_h, ex_h, denomp_h,
           dstb, logb, mxred, denb, slab, mx_sh):
    c = lax.axis_index("c")
    s = lax.axis_index("s")
    w = _wid(c, s)
    e0 = w * EPT
    n0 = s * NPS
    pltpu.sync_copy(mxp_h.at[:, pl.ds(n0, NPS)], slab)

    def red_body(j, _):
        sl = pl.ds(j * 16, 16)

        def rr(r, acc):
            return jnp.maximum(acc, slab[r, sl])

        acc = lax.fori_loop(1, NW, rr, slab[0, sl])
        mxred[pl.ds(n0 + j * 16, 16)] = acc
        return 0

    lax.fori_loop(0, NPS // 16, red_body, 0)
    pltpu.sync_copy(mxred.at[pl.ds(n0, NPS)], mx_sh.at[pl.ds(n0, NPS)])
    plsc.subcore_barrier()
    pltpu.sync_copy(mx_sh, mxred)

    pltpu.sync_copy(dst_h.at[pl.ds(e0, EPT)], dstb)
    pltpu.sync_copy(logits_h.at[pl.ds(e0, EPT)], logb)

    zero16f = jnp.zeros((16,), jnp.float32)

    def init_body(i, _):
        denb[pl.ds(i * 16, 16)] = zero16f
        return 0

    lax.fori_loop(0, NP // 16, init_body, 0)

    def grp_body(g, _):
        sl = pl.ds(g * 16, 16)
        dv = dstb[sl]
        lg = logb[sl]
        mxd = plsc.load_gather(mxred, [dv])
        exv = jnp.exp(lg - mxd)
        logb[sl] = exv
        _seg_update(dv, exv, denb, "add")
        return 0

    lax.fori_loop(0, NGRP, grp_body, 0)

    pltpu.sync_copy(logb, ex_h.at[pl.ds(e0, EPT)])
    pltpu.sync_copy(denb, denomp_h.at[w])


@functools.partial(
    pl.kernel,
    out_type=jax.ShapeDtypeStruct((NSC, NP, HH), jnp.float32),
    mesh=_mesh,
    compiler_params=pltpu.CompilerParams(needs_layout_passes=False,
                                         use_tc_tiling_on_sc=False),
    scratch_types=[
        pltpu.VMEM((EPTP,), jnp.int32),
        pltpu.VMEM((EPTP,), jnp.int32),
        pltpu.VMEM((NCHUNK, CH), jnp.int32),
        pltpu.VMEM((EPTP,), jnp.float32),
        pltpu.VMEM((NP,), jnp.float32),
        pltpu.VMEM((NW, NPS), jnp.float32),
        pltpu.VMEM((CH, HH), jnp.float32),
        pltpu.VMEM_SHARED((NP,), jnp.float32),
        pltpu.VMEM_SHARED((NP, HH), jnp.float32),
        pltpu.SemaphoreType.DMA,
    ],
)
def _sc_a5(src_h, dst_h, et_h, ex_h, denomp_h, hwe_h, part_h,
           stage, etb, dstb2, alb, denred, slab, rows, den_sh, acc_sh, sem):
    c = lax.axis_index("c")
    s = lax.axis_index("s")
    w = _wid(c, s)
    e0 = w * EPT
    n0 = s * NPS

    pltpu.sync_copy(denomp_h.at[:, pl.ds(n0, NPS)], slab)

    def red_body(j, _):
        sl = pl.ds(j * 16, 16)

        def rr(r, acc):
            return acc + slab[r, sl]

        acc = lax.fori_loop(1, NW, rr, slab[0, sl])
        denred[pl.ds(n0 + j * 16, 16)] = acc
        return 0

    lax.fori_loop(0, NPS // 16, red_body, 0)
    pltpu.sync_copy(denred.at[pl.ds(n0, NPS)], den_sh.at[pl.ds(n0, NPS)])
    plsc.subcore_barrier()
    pltpu.sync_copy(den_sh, denred)

    zero16 = jnp.zeros((16,), jnp.int32)
    pltpu.sync_copy(dst_h.at[pl.ds(e0, EPT)], stage.at[pl.ds(0, EPT)])

    def pad_i(t, _):
        stage[pl.ds(EPT + t * 16, 16)] = zero16
        return 0

    lax.fori_loop(0, (EPTP - EPT) // 16, pad_i, 0)

    def dst_body(j, _):
        for k in range(8):
            dstb2[j, pl.ds(k * 16, 16)] = stage[pl.ds(j * CH + k * 16, 16)]
        return 0

    lax.fori_loop(0, NCHUNK, dst_body, 0)

    pltpu.sync_copy(et_h.at[pl.ds(e0, EPT)], etb.at[pl.ds(0, EPT)])
    pltpu.sync_copy(src_h.at[pl.ds(e0, EPT)], stage.at[pl.ds(0, EPT)])
    lax.fori_loop(0, (EPTP - EPT) // 16, pad_i, 0)

    def pad_e(t, _):
        etb[pl.ds(EPT + t * 16, 16)] = zero16
        return 0

    lax.fori_loop(0, (EPTP - EPT) // 16, pad_e, 0)

    def comb_body(g, _):
        sl = pl.ds(g * 16, 16)
        stage[sl] = etb[sl] * NP + stage[sl]
        return 0

    lax.fori_loop(0, NGRPP, comb_body, 0)

    pltpu.sync_copy(ex_h.at[pl.ds(e0, EPT)], alb.at[pl.ds(0, EPT)])
    zero16f = jnp.zeros((16,), jnp.float32)

    def pad_a(t, _):
        alb[pl.ds(EPT + t * 16, 16)] = zero16f
        return 0

    lax.fori_loop(0, (EPTP - EPT) // 16, pad_a, 0)

    def al_body(j, _):
        for k in range(8):
            sl = pl.ds(j * CH + k * 16, 16)
            dv = dstb2[j, pl.ds(k * 16, 16)]
            dn = plsc.load_gather(denred, [dv])
            alb[sl] = alb[sl] / (dn + 1e-16)
        return 0

    lax.fori_loop(0, NCHUNK, al_body, 0)

    def zrow_body(r, _):
        for k in range(HH // 16):
            rows[r, pl.ds(k * 16, 16)] = zero16f
        return 0

    lax.fori_loop(0, CH, zrow_body, 0)
    for k in range(NPS // CH):
        pltpu.sync_copy(rows, acc_sh.at[pl.ds(n0 + k * CH, CH), :])
    plsc.subcore_barrier()

    def chunk_body(j, _):
        idx = stage.at[pl.ds(j * CH, CH)]
        pltpu.async_copy(hwe_h.at[idx], rows, sem).wait()

        def e_body(e16, _):
            av = alb[pl.ds(j * CH + e16 * 16, 16)]
            for t in range(16):
                avb = jnp.full((16,), av[t], jnp.float32)
                for k in range(HH // 16):
                    sl = pl.ds(k * 16, 16)
                    rows[e16 * 16 + t, sl] = rows[e16 * 16 + t, sl] * avb
            return 0

        lax.fori_loop(0, CH // 16, e_body, 0)
        pltpu.sync_copy(rows, acc_sh.at[dstb2.at[j]], add=True)
        return 0

    lax.fori_loop(0, NCHUNK, chunk_body, 0)
    plsc.subcore_barrier()
    pltpu.sync_copy(acc_sh.at[pl.ds(n0, NPS), :],
                    part_h.at[c, pl.ds(n0, NPS), :])


@functools.partial(
    pl.kernel,
    out_type=jax.ShapeDtypeStruct((NW, 16), jnp.float32),
    mesh=_mesh,
    compiler_params=pltpu.CompilerParams(needs_layout_passes=False),
    scratch_types=[pltpu.VMEM((16,), jnp.float32)],
)
def _dummy(x_h, o_h, buf):
    c = lax.axis_index("c")
    s = lax.axis_index("s")
    w = _wid(c, s)
    pltpu.sync_copy(x_h.at[w], buf)
    buf[...] = buf[...] * 2.0
    pltpu.sync_copy(buf, o_h.at[w])


def kernel(x, edge_index, edge_type, batch, W0, as0, ad0, We0, W1, as1, ad1,
           We1, W2, as2, ad2, We2, Wout, bout, Wv1, bv1, Wf, bf):
    f32 = jnp.float32
    xp = jnp.zeros((NP, H), f32).at[:N].set(x)
    src = edge_index[0].astype(jnp.int32)
    dst = edge_index[1].astype(jnp.int32)
    et = edge_type.astype(jnp.int32)
    batch_p = jnp.full((NP,), G, jnp.int32).at[:N].set(batch.astype(jnp.int32))
    b3 = batch_p.reshape(NBLK, 1, BLK)

    params = [(W0, as0, ad0, We0), (W1, as1, ad1, We1), (W2, as2, ad2, We2)]
    part = None
    for l, (W, a_s, a_d, We) in enumerate(params):
        asad = jnp.stack([a_s, a_d], axis=1)
        if l == 0:
            hwe0, hwe1, sd, es = _tc_layer_first(xp, W, asad, We)
        else:
            hwe0, hwe1, sd, es = _tc_layer_next(
                part[0][0], part[0][1], part[1][0], part[1][1], W, asad, We)
        logits, mxp = _sc_a1(src, dst, et, sd.reshape(2 * NP),
                             es.reshape(4 * NT))
        ex, denomp = _sc_a3(dst, logits, mxp)
        pa = _sc_a5(src, dst, et, ex, denomp, hwe0.reshape(NT * NP, HH))
        pb = _sc_a5(src, dst, et, ex, denomp, hwe1.reshape(NT * NP, HH))
        part = (pa, pb)

    wfp = jnp.pad(Wf, ((0, 0), (0, 7)))
    bfp = jnp.pad(bf, (0, 7)).reshape(1, 8)
    o = _tc_head(part[0][0], part[0][1], part[1][0], part[1][1], b3, Wout,
                 bout.reshape(1, 256), Wv1, bv1.reshape(1, 256), wfp, bfp)
    z = jnp.zeros((NW, 16), jnp.float32) + o[0, 0]
    for _ in range(6):
        z = _dummy(z)
    return o[:, 0] + 0.0 * z[0, 0]


# speedup vs baseline: 1.7755x; 1.7755x over previous
import functools

import jax
import jax.numpy as jnp
from jax import lax
from jax.experimental import pallas as pl
from jax.experimental.pallas import tpu as pltpu
from jax.experimental.pallas import tpu_sc as plsc

N = 10000
NP = 10240
E = 320000
H = 128
NT = 4
G = 64

NSC = 2
NSUB = 16
NW = NSC * NSUB
EPT = E // NW
EPTP = 10240
CH = 64
NCHUNK = EPTP // CH
NPS = NP // NSUB
NGRP = EPT // 16
NGRPP = EPTP // 16
HH = H // 2
BLK = 1024
NBLK = NP // BLK

_mesh = plsc.VectorSubcoreMesh(core_axis_name="c", subcore_axis_name="s",
                               num_cores=NSC, num_subcores=NSUB)

_IOTA = lambda: lax.broadcasted_iota(jnp.int32, (16,), 0)


def _vgather(v, i):
    return v.at[i].get(mode="promise_in_bounds")


def _seg_update(dv, val, outref, op):
    iota = _IOTA()
    ks, vs = plsc.sort_key_val(dv, val)
    for sh in (1, 2, 4, 8):
        idxm = jnp.maximum(iota - sh, 0)
        kprev = _vgather(ks, idxm)
        vprev = _vgather(vs, idxm)
        valid = (iota >= sh) & (kprev == ks)
        if op == "max":
            vs = jnp.where(valid, jnp.maximum(vs, vprev), vs)
        else:
            vs = vs + jnp.where(valid, vprev, jnp.float32(0.0))
    knext = _vgather(ks, jnp.minimum(iota + 1, 15))
    runend = (knext != ks) | (iota == 15)
    old = plsc.load_gather(outref, [ks])
    newv = jnp.maximum(old, vs) if op == "max" else old + vs
    plsc.store_scatter(outref, [ks], newv, mask=runend)


def _wid(c, s):
    return c * NSUB + s


def _tc_layer_body(first, *refs):
    if first:
        x0_ref, w_ref, asad_ref, we_ref, hwe0_ref, hwe1_ref, sd_ref, es_ref = refs
        h = x0_ref[...]
    else:
        (pa0_ref, pa1_ref, pb0_ref, pb1_ref, w_ref, asad_ref, we_ref,
         hwe0_ref, hwe1_ref, sd_ref, es_ref) = refs
        h = jnp.concatenate(
            [jnp.maximum(pa0_ref[...] + pa1_ref[...], 0.0),
             jnp.maximum(pb0_ref[...] + pb1_ref[...], 0.0)], axis=1)
    i = pl.program_id(0)
    hw = jnp.dot(h, w_ref[...], preferred_element_type=jnp.float32)
    we = we_ref[...]
    hwe0_ref[...] = hw[None, :, :HH] + we[:, None, :HH]
    hwe1_ref[...] = hw[None, :, HH:] + we[:, None, HH:]
    sd_ref[...] = jnp.dot(hw, asad_ref[...], preferred_element_type=jnp.float32)

    @pl.when(i == 0)
    def _():
        esd = jnp.dot(we_ref[...], asad_ref[...],
                      preferred_element_type=jnp.float32)
        es_ref[...] = jnp.concatenate(
            [esd, jnp.zeros((NT, 2), jnp.float32)], axis=0)


def _make_tc_layer(first):
    if first:
        in_specs = [pl.BlockSpec((BLK, H), lambda i: (i, 0))]
    else:
        in_specs = [pl.BlockSpec((BLK, HH), lambda i: (i, 0))] * 4

    def run(*args):
        return pl.pallas_call(
            functools.partial(_tc_layer_body, first),
            grid=(NBLK,),
            in_specs=in_specs + [
                pl.BlockSpec((H, H), lambda i: (0, 0)),
                pl.BlockSpec((H, 2), lambda i: (0, 0)),
                pl.BlockSpec((NT, H), lambda i: (0, 0)),
            ],
            out_specs=[
                pl.BlockSpec((NT, BLK, HH), lambda i: (0, i, 0)),
                pl.BlockSpec((NT, BLK, HH), lambda i: (0, i, 0)),
                pl.BlockSpec((BLK, 2), lambda i: (i, 0)),
                pl.BlockSpec((2 * NT, 2), lambda i: (0, 0)),
            ],
            out_shape=[
                jax.ShapeDtypeStruct((NT, NP, HH), jnp.float32),
                jax.ShapeDtypeStruct((NT, NP, HH), jnp.float32),
                jax.ShapeDtypeStruct((NP, 2), jnp.float32),
                jax.ShapeDtypeStruct((2 * NT, 2), jnp.float32),
            ],
        )(*args)

    return run


_tc_layer_first = _make_tc_layer(True)
_tc_layer_next = _make_tc_layer(False)


def _tc_head_body(pa0_ref, pa1_ref, pb0_ref, pb1_ref, b3_ref, wout_ref,
                  bout_ref, wv1_ref, bv1_ref, wf_ref, bf_ref, o_ref, g_acc):
    i = pl.program_id(0)

    @pl.when(i == 0)
    def _():
        g_acc[...] = jnp.zeros_like(g_acc)

    h = jnp.concatenate(
        [jnp.maximum(pa0_ref[...] + pa1_ref[...], 0.0),
         jnp.maximum(pb0_ref[...] + pb1_ref[...], 0.0)], axis=1)
    ho = jnp.dot(h, wout_ref[...], preferred_element_type=jnp.float32)
    ho = ho + bout_ref[0, :]
    b = b3_ref[0, 0, :]
    ids = lax.broadcasted_iota(jnp.int32, (G, BLK), 0)
    m = jnp.where(b[None, :] == ids, 1.0, 0.0)
    g_acc[...] += jnp.dot(m, ho, preferred_element_type=jnp.float32)

    @pl.when(i == NBLK - 1)
    def _():
        xx = jnp.maximum(g_acc[...], 0.0)
        xx = jnp.dot(xx, wv1_ref[...], preferred_element_type=jnp.float32)
        xx = jnp.maximum(xx + bv1_ref[0, :], 0.0)
        o_ref[...] = jnp.dot(xx, wf_ref[...],
                             preferred_element_type=jnp.float32) + bf_ref[0, :]


def _tc_head(pa0, pa1, pb0, pb1, b3, wout, bout, wv1, bv1, wfp, bfp):
    return pl.pallas_call(
        _tc_head_body,
        grid=(NBLK,),
        in_specs=[
            pl.BlockSpec((BLK, HH), lambda i: (i, 0)),
            pl.BlockSpec((BLK, HH), lambda i: (i, 0)),
            pl.BlockSpec((BLK, HH), lambda i: (i, 0)),
            pl.BlockSpec((BLK, HH), lambda i: (i, 0)),
            pl.BlockSpec((1, 1, BLK), lambda i: (i, 0, 0)),
            pl.BlockSpec((H, 256), lambda i: (0, 0)),
            pl.BlockSpec((1, 256), lambda i: (0, 0)),
            pl.BlockSpec((256, 256), lambda i: (0, 0)),
            pl.BlockSpec((1, 256), lambda i: (0, 0)),
            pl.BlockSpec((256, 8), lambda i: (0, 0)),
            pl.BlockSpec((1, 8), lambda i: (0, 0)),
        ],
        out_specs=pl.BlockSpec((G, 8), lambda i: (0, 0)),
        out_shape=jax.ShapeDtypeStruct((G, 8), jnp.float32),
        scratch_shapes=[pltpu.VMEM((G, 256), jnp.float32)],
    )(pa0, pa1, pb0, pb1, b3, wout, bout, wv1, bv1, wfp, bfp)


U = 5
NBUF = 4
NJ4 = NCHUNK // NBUF


@functools.partial(
    pl.kernel,
    out_type=(jax.ShapeDtypeStruct((E,), jnp.float32),
              jax.ShapeDtypeStruct((NW, NP), jnp.float32)),
    mesh=_mesh,
    compiler_params=pltpu.CompilerParams(needs_layout_passes=False),
    scratch_types=[
        pltpu.VMEM((EPT,), jnp.int32),
        pltpu.VMEM((EPT,), jnp.int32),
        pltpu.VMEM((EPT,), jnp.int32),
        pltpu.VMEM((2 * NP,), jnp.float32),
        pltpu.VMEM((4 * NT,), jnp.float32),
        pltpu.VMEM((EPT,), jnp.float32),
    ] + [pltpu.VMEM((NP,), jnp.float32)] * U,
)
def _sc_a1(src_h, dst_h, et_h, sd_h, es_h, logits_h, mxp_h,
           srcb, dstb, etb, sdb, esb, logb, *mxbs):
    c = lax.axis_index("c")
    s = lax.axis_index("s")
    w = _wid(c, s)
    e0 = w * EPT
    pltpu.sync_copy(src_h.at[pl.ds(e0, EPT)], srcb)
    pltpu.sync_copy(dst_h.at[pl.ds(e0, EPT)], dstb)
    pltpu.sync_copy(et_h.at[pl.ds(e0, EPT)], etb)
    pltpu.sync_copy(sd_h, sdb)
    pltpu.sync_copy(es_h, esb)

    neginf = jnp.full((16,), -jnp.inf, jnp.float32)

    def init_body(i, _):
        for k in range(8):
            sl = pl.ds((i * 8 + k) * 16, 16)
            for b in range(U):
                mxbs[b][sl] = neginf
        return 0

    lax.fori_loop(0, NP // 128, init_body, 0)

    def grp_body(g5, _):
        for u in range(U):
            sl = pl.ds((g5 * U + u) * 16, 16)
            sv = srcb[sl]
            dv = dstb[sl]
            ev = etb[sl]
            s16 = plsc.load_gather(sdb, [sv * 2])
            d16 = plsc.load_gather(sdb, [dv * 2 + 1])
            e16 = plsc.load_gather(esb, [ev * 2])
            z = s16 + d16 + e16
            lg = jnp.maximum(z, 0.2 * z)
            logb[sl] = lg
            _seg_update(dv, lg, mxbs[u], "max")
        return 0

    lax.fori_loop(0, NGRP // U, grp_body, 0)

    def mrg_body(i, _):
        for k in range(4):
            sl = pl.ds((i * 4 + k) * 16, 16)
            m = mxbs[0][sl]
            for b in range(1, U):
                m = jnp.maximum(m, mxbs[b][sl])
            mxbs[0][sl] = m
        return 0

    lax.fori_loop(0, NP // 64, mrg_body, 0)

    pltpu.sync_copy(logb, logits_h.at[pl.ds(e0, EPT)])
    pltpu.sync_copy(mxbs[0], mxp_h.at[w])


@functools.partial(
    pl.kernel,
    out_type=(jax.ShapeDtypeStruct((E,), jnp.float32),
              jax.ShapeDtypeStruct((NW, NP), jnp.float32)),
    mesh=_mesh,
    compiler_params=pltpu.CompilerParams(needs_layout_passes=False),
    scratch_types=[
        pltpu.VMEM((EPT,), jnp.int32),
        pltpu.VMEM((EPT,), jnp.float32),
        pltpu.VMEM((NP,), jnp.float32),
        pltpu.VMEM((NW, NPS), jnp.float32),
        pltpu.VMEM_SHARED((NP,), jnp.float32),
    ] + [pltpu.VMEM((NP,), jnp.float32)] * U,
)
def _sc_a3(dst_h, logits_h, mxp_h, ex_h, denomp_h,
           dstb, logb, mxred, slab, mx_sh, *denbs):
    c = lax.axis_index("c")
    s = lax.axis_index("s")
    w = _wid(c, s)
    e0 = w * EPT
    n0 = s * NPS
    pltpu.sync_copy(mxp_h.at[:, pl.ds(n0, NPS)], slab)

    def red_body(j, _):
        sl = pl.ds(j * 16, 16)
        acc = slab[0, sl]
        for r in range(1, NW):
            acc = jnp.maximum(acc, slab[r, sl])
        mxred[pl.ds(n0 + j * 16, 16)] = acc
        return 0

    lax.fori_loop(0, NPS // 16, red_body, 0)
    pltpu.sync_copy(mxred.at[pl.ds(n0, NPS)], mx_sh.at[pl.ds(n0, NPS)])
    plsc.subcore_barrier()
    pltpu.sync_copy(mx_sh, mxred)

    pltpu.sync_copy(dst_h.at[pl.ds(e0, EPT)], dstb)
    pltpu.sync_copy(logits_h.at[pl.ds(e0, EPT)], logb)

    zero16f = jnp.zeros((16,), jnp.float32)

    def init_body(i, _):
        for k in range(8):
            sl = pl.ds((i * 8 + k) * 16, 16)
            for b in range(U):
                denbs[b][sl] = zero16f
        return 0

    lax.fori_loop(0, NP // 128, init_body, 0)

    def grp_body(g5, _):
        for u in range(U):
            sl = pl.ds((g5 * U + u) * 16, 16)
            dv = dstb[sl]
            lg = logb[sl]
            mxd = plsc.load_gather(mxred, [dv])
            exv = jnp.exp(lg - mxd)
            logb[sl] = exv
            _seg_update(dv, exv, denbs[u], "add")
        return 0

    lax.fori_loop(0, NGRP // U, grp_body, 0)

    def mrg_body(i, _):
        for k in range(4):
            sl = pl.ds((i * 4 + k) * 16, 16)
            m = denbs[0][sl]
            for b in range(1, U):
                m = m + denbs[b][sl]
            denbs[0][sl] = m
        return 0

    lax.fori_loop(0, NP // 64, mrg_body, 0)

    pltpu.sync_copy(logb, ex_h.at[pl.ds(e0, EPT)])
    pltpu.sync_copy(denbs[0], denomp_h.at[w])


@functools.partial(
    pl.kernel,
    out_type=(jax.ShapeDtypeStruct((NSC, NP, HH), jnp.float32),
              jax.ShapeDtypeStruct((NSC, NP, HH), jnp.float32)),
    mesh=_mesh,
    compiler_params=pltpu.CompilerParams(needs_layout_passes=False,
                                         use_tc_tiling_on_sc=False),
    scratch_types=[
        pltpu.VMEM((EPTP,), jnp.int32),
        pltpu.VMEM((NCHUNK, CH), jnp.int32),
        pltpu.VMEM((EPTP,), jnp.float32),
        pltpu.VMEM((NP,), jnp.float32),
        pltpu.VMEM((NW, NPS // 4), jnp.float32),
    ] + [pltpu.VMEM((CH, HH), jnp.float32)] * (2 * NBUF) + [
        pltpu.VMEM_SHARED((NP,), jnp.float32),
        pltpu.VMEM_SHARED((NP, HH), jnp.float32),
    ] + [pltpu.SemaphoreType.DMA] * (2 * NBUF),
)
def _sc_a5(comb_h, dst_h, ex_h, denomp_h, hwe0_h, hwe1_h, p0_h, p1_h,
           stage, dstb2, alb, denred, slab, *bufs):
    rows = list(bufs[:NBUF])
    sbuf = list(bufs[NBUF:2 * NBUF])
    den_sh = bufs[2 * NBUF]
    acc_sh = bufs[2 * NBUF + 1]
    gsem = list(bufs[2 * NBUF + 2:2 * NBUF + 2 + NBUF])
    ssem = list(bufs[2 * NBUF + 2 + NBUF:2 * NBUF + 2 + 2 * NBUF])
    c = lax.axis_index("c")
    s = lax.axis_index("s")
    w = _wid(c, s)
    e0 = w * EPT
    n0 = s * NPS
    zero16 = jnp.zeros((16,), jnp.int32)
    zero16f = jnp.zeros((16,), jnp.float32)

    QS = NPS // 4
    for cc in range(4):
        pltpu.sync_copy(denomp_h.at[:, pl.ds(n0 + cc * QS, QS)], slab)

        def red_body(j, _):
            sl = pl.ds(j * 16, 16)
            acc = slab[0, sl]
            for r in range(1, NW):
                acc = acc + slab[r, sl]
            denred[pl.ds(n0 + cc * QS + j * 16, 16)] = acc
            return 0

        lax.fori_loop(0, QS // 16, red_body, 0)
    pltpu.sync_copy(denred.at[pl.ds(n0, NPS)], den_sh.at[pl.ds(n0, NPS)])
    plsc.subcore_barrier()
    pltpu.sync_copy(den_sh, denred)

    pltpu.sync_copy(dst_h.at[pl.ds(e0, EPT)], stage.at[pl.ds(0, EPT)])
    for t in range((EPTP - EPT) // 16):
        stage[pl.ds(EPT + t * 16, 16)] = zero16

    def dst_body(j, _):
        for k in range(CH // 16):
            dstb2[j, pl.ds(k * 16, 16)] = stage[pl.ds(j * CH + k * 16, 16)]
        return 0

    lax.fori_loop(0, NCHUNK, dst_body, 0)

    pltpu.sync_copy(comb_h.at[pl.ds(e0, EPT)], stage.at[pl.ds(0, EPT)])
    for t in range((EPTP - EPT) // 16):
        stage[pl.ds(EPT + t * 16, 16)] = zero16

    pltpu.sync_copy(ex_h.at[pl.ds(e0, EPT)], alb.at[pl.ds(0, EPT)])
    for t in range((EPTP - EPT) // 16):
        alb[pl.ds(EPT + t * 16, 16)] = zero16f

    def al_body(j, _):
        for k in range(CH // 16):
            sl = pl.ds(j * CH + k * 16, 16)
            dv = dstb2[j, pl.ds(k * 16, 16)]
            dn = plsc.load_gather(denred, [dv])
            alb[sl] = alb[sl] / (dn + 1e-16)
        return 0

    lax.fori_loop(0, NCHUNK, al_body, 0)

    for hwe, out in ((hwe0_h, p0_h), (hwe1_h, p1_h)):
        def z_body(r, _):
            for k in range(HH // 16):
                rows[0][r, pl.ds(k * 16, 16)] = zero16f
            return 0

        lax.fori_loop(0, CH, z_body, 0)
        for k in range(NPS // CH):
            pltpu.sync_copy(rows[0], acc_sh.at[pl.ds(n0 + k * CH, CH), :])
        plsc.subcore_barrier()

        for b in range(NBUF):
            pltpu.async_copy(hwe.at[stage.at[pl.ds(b * CH, CH)]],
                             rows[b], gsem[b])

        def j4_body(j4, _):
            for b in range(NBUF):
                j = j4 * NBUF + b
                pltpu.make_async_copy(
                    hwe.at[stage.at[pl.ds(j * CH, CH)]], rows[b],
                    gsem[b]).wait()

                @pl.when(j4 > 0)
                def _drain():
                    pltpu.make_async_copy(
                        sbuf[b], acc_sh.at[dstb2.at[j]], ssem[b]).wait()

                def scale16(e16, _):
                    av = alb[pl.ds(j * CH + e16 * 16, 16)]
                    for t in range(16):
                        avb = jnp.full((16,), av[t], jnp.float32)
                        for k in range(HH // 16):
                            sl = pl.ds(k * 16, 16)
                            sbuf[b][e16 * 16 + t, sl] = (
                                rows[b][e16 * 16 + t, sl] * avb)
                    return 0

                lax.fori_loop(0, CH // 16, scale16, 0)

                @pl.when(j4 < NJ4 - 1)
                def _refill():
                    pltpu.async_copy(
                        hwe.at[stage.at[pl.ds((j + NBUF) * CH, CH)]],
                        rows[b], gsem[b])

                pltpu.async_copy(sbuf[b], acc_sh.at[dstb2.at[j]], ssem[b],
                                 add=True)
            return 0

        lax.fori_loop(0, NJ4, j4_body, 0)
        for b in range(NBUF):
            pltpu.make_async_copy(
                sbuf[b],
                acc_sh.at[dstb2.at[NCHUNK - NBUF + b]], ssem[b]).wait()
        plsc.subcore_barrier()
        pltpu.sync_copy(acc_sh.at[pl.ds(n0, NPS), :],
                        out.at[c, pl.ds(n0, NPS), :])
        plsc.subcore_barrier()


def kernel(x, edge_index, edge_type, batch, W0, as0, ad0, We0, W1, as1, ad1,
           We1, W2, as2, ad2, We2, Wout, bout, Wv1, bv1, Wf, bf):
    f32 = jnp.float32
    xp = jnp.zeros((NP, H), f32).at[:N].set(x)
    src = edge_index[0].astype(jnp.int32)
    dst = edge_index[1].astype(jnp.int32)
    et = edge_type.astype(jnp.int32)
    comb = et * NP + src
    batch_p = jnp.full((NP,), G, jnp.int32).at[:N].set(batch.astype(jnp.int32))
    b3 = batch_p.reshape(NBLK, 1, BLK)

    params = [(W0, as0, ad0, We0), (W1, as1, ad1, We1), (W2, as2, ad2, We2)]
    part = None
    for l, (W, a_s, a_d, We) in enumerate(params):
        asad = jnp.stack([a_s, a_d], axis=1)
        if l == 0:
            hwe0, hwe1, sd, es = _tc_layer_first(xp, W, asad, We)
        else:
            hwe0, hwe1, sd, es = _tc_layer_next(
                part[0][0], part[0][1], part[1][0], part[1][1], W, asad, We)
        logits, mxp = _sc_a1(src, dst, et, sd.reshape(2 * NP),
                             es.reshape(4 * NT))
        ex, denomp = _sc_a3(dst, logits, mxp)
        pa, pb = _sc_a5(comb, dst, ex, denomp,
                        hwe0.reshape(NT * NP, HH),
                        hwe1.reshape(NT * NP, HH))
        part = (pa, pb)

    wfp = jnp.pad(Wf, ((0, 0), (0, 7)))
    bfp = jnp.pad(bf, (0, 7)).reshape(1, 8)
    o = _tc_head(part[0][0], part[0][1], part[1][0], part[1][1], b3, Wout,
                 bout.reshape(1, 256), Wv1, bv1.reshape(1, 256), wfp, bfp)
    return o[:, 0]
